# baseline (speedup 1.0000x reference)
import jax
import jax.numpy as jnp
from jax import lax
from jax.experimental import pallas as pl
from jax.experimental.pallas import tpu as pltpu
from jax.experimental.pallas import tpu_sc as plsc

_VOCAB = 1000000
_EMBED = 32
_MAXLEN = 50
_HIDDEN = 128
_OUT = 1
_BATCH = 4096

_NC = 2
_NS = 16
_NW = _NC * _NS
_TOTAL = _BATCH * _MAXLEN
_PER_W = _TOTAL // _NW
_CHUNK = 128
_NCHUNK = _PER_W // _CHUNK

_TBLK = 8192


def _twiden_body(eT_ref, o_ref):
    eye = jnp.eye(_EMBED, dtype=jnp.float32)
    t = lax.dot_general(eT_ref[...], eye, (((0,), (0,)), ((), ())),
                        preferred_element_type=jnp.float32)
    o_ref[:, 0:_EMBED] = t.astype(jnp.bfloat16)


def _tc_twiden(embT):
    grid = (-(-_VOCAB // _TBLK),)
    return pl.pallas_call(
        _twiden_body,
        grid=grid,
        in_specs=[pl.BlockSpec((_EMBED, _TBLK), lambda i: (0, i))],
        out_specs=pl.BlockSpec((_TBLK, 128), lambda i: (i, 0)),
        out_shape=jax.ShapeDtypeStruct((_VOCAB, 128), jnp.bfloat16),
    )(embT)


def _gather_body(idx_hbm, embed_hbm, out_hbm, idx_v, rows0, rows1,
                 sem0, sem1):
    wid = lax.axis_index("s") * _NC + lax.axis_index("c")
    base = wid * _PER_W
    pltpu.sync_copy(idx_hbm.at[wid], idx_v)
    rows = (rows0, rows1)
    sems = (sem0, sem1)

    pltpu.async_copy(embed_hbm.at[idx_v.at[0]], rows[0], sems[0])
    for j in range(_NCHUNK):
        if j + 1 < _NCHUNK:
            pltpu.async_copy(embed_hbm.at[idx_v.at[j + 1]],
                             rows[(j + 1) % 2], sems[(j + 1) % 2])
        pltpu.make_async_copy(embed_hbm.at[idx_v.at[j]], rows[j % 2],
                              sems[j % 2]).wait()
        pltpu.sync_copy(rows[j % 2].at[:, pl.ds(0, _EMBED)],
                        out_hbm.at[pl.ds(base + j * _CHUNK, _CHUNK)])


def _sc_gather(idx3, embed):
    mesh = plsc.VectorSubcoreMesh(core_axis_name="c", subcore_axis_name="s")
    fn = pl.kernel(
        _gather_body,
        mesh=mesh,
        out_type=jax.ShapeDtypeStruct((_TOTAL, _EMBED), jnp.bfloat16),
        scratch_types=[
            pltpu.VMEM((_NCHUNK, _CHUNK), jnp.int32),
            pltpu.VMEM((_CHUNK, 128), jnp.bfloat16),
            pltpu.VMEM((_CHUNK, 128), jnp.bfloat16),
            pltpu.SemaphoreType.DMA,
            pltpu.SemaphoreType.DMA,
        ],
        compiler_params=pltpu.CompilerParams(use_tc_tiling_on_sc=False),
    )
    return fn(idx3, embed)


def _mlp_body(x_ref, w1_ref, b1_ref, w2_ref, b2_ref, w3_ref, b3_ref, y_ref):
    x = x_ref[...].astype(jnp.float32)
    h = lax.dot_general(x, w1_ref[...], (((1,), (1,)), ((), ())),
                        preferred_element_type=jnp.float32)
    h = jnp.maximum(h + b1_ref[...], 0.0)
    h = lax.dot_general(h, w2_ref[...], (((1,), (1,)), ((), ())),
                        preferred_element_type=jnp.float32)
    h = jnp.maximum(h + b2_ref[...], 0.0)
    y = lax.dot_general(w3_ref[...], h, (((1,), (1,)), ((), ())),
                        preferred_element_type=jnp.float32)
    y_ref[...] = y + b3_ref[0, 0]


def _tc_mlp(x, W1, b1, W2, b2, W3, b3):
    bb = 1024
    grid = (_BATCH // bb,)
    return pl.pallas_call(
        _mlp_body,
        grid=grid,
        in_specs=[
            pl.BlockSpec((bb, _MAXLEN * _EMBED), lambda i: (i, 0)),
            pl.BlockSpec((_HIDDEN, _MAXLEN * _EMBED), lambda i: (0, 0)),
            pl.BlockSpec((1, _HIDDEN), lambda i: (0, 0)),
            pl.BlockSpec((_HIDDEN, _HIDDEN), lambda i: (0, 0)),
            pl.BlockSpec((1, _HIDDEN), lambda i: (0, 0)),
            pl.BlockSpec((_OUT, _HIDDEN), lambda i: (0, 0)),
            pl.BlockSpec(memory_space=pltpu.SMEM),
        ],
        out_specs=pl.BlockSpec((_OUT, bb), lambda i: (0, i)),
        out_shape=jax.ShapeDtypeStruct((_OUT, _BATCH), jnp.float32),
    )(x, W1, b1, W2, b2, W3, b3)


def kernel(input, embed, W1, b1, W2, b2, W3, b3):
    idx3 = input.reshape(_NW, _NCHUNK, _CHUNK)
    table = _tc_twiden(embed.T)
    rows = _sc_gather(idx3, table)
    x = rows.reshape(_BATCH, _MAXLEN * _EMBED)
    y = _tc_mlp(x, W1, b1.reshape(1, -1), W2, b2.reshape(1, -1),
                W3, b3.reshape(1, -1))
    return y.reshape(_BATCH, _OUT)


# speedup vs baseline: 3.6491x; 3.6491x over previous
import jax
import jax.numpy as jnp
from jax import lax
from jax.experimental import pallas as pl
from jax.experimental.pallas import tpu as pltpu
from jax.experimental.pallas import tpu_sc as plsc

_VOCAB = 1000000
_EMBED = 32
_MAXLEN = 50
_HIDDEN = 128
_OUT = 1
_BATCH = 4096

_NC = 2
_NS = 16
_NW = _NC * _NS
_TOTAL = _BATCH * _MAXLEN
_PER_W = _TOTAL // _NW
_CHUNK = 128
_NCHUNK = _PER_W // _CHUNK

_TBLK = 16384


def _twiden_body(eT_ref, o_ref):
    eye = jnp.eye(_EMBED, dtype=jnp.float32)
    o_ref[:, 0:_EMBED] = lax.dot_general(
        eT_ref[...], eye, (((0,), (0,)), ((), ())),
        preferred_element_type=jnp.float32)


def _tc_twiden(embT):
    grid = (-(-_VOCAB // _TBLK),)
    return pl.pallas_call(
        _twiden_body,
        grid=grid,
        in_specs=[pl.BlockSpec((_EMBED, _TBLK), lambda i: (0, i))],
        out_specs=pl.BlockSpec((_TBLK, 128), lambda i: (i, 0)),
        out_shape=jax.ShapeDtypeStruct((_VOCAB, 128), jnp.float32),
    )(embT)


def _gather_body(idx_hbm, embed_hbm, out_hbm, idx_v, rows0, rows1,
                 sem0, sem1):
    wid = lax.axis_index("s") * _NC + lax.axis_index("c")
    base = wid * _PER_W
    pltpu.sync_copy(idx_hbm.at[wid], idx_v)
    rows = (rows0, rows1)
    sems = (sem0, sem1)

    pltpu.async_copy(embed_hbm.at[idx_v.at[0]], rows[0], sems[0])
    for j in range(_NCHUNK):
        if j + 1 < _NCHUNK:
            pltpu.async_copy(embed_hbm.at[idx_v.at[j + 1]],
                             rows[(j + 1) % 2], sems[(j + 1) % 2])
        pltpu.make_async_copy(embed_hbm.at[idx_v.at[j]], rows[j % 2],
                              sems[j % 2]).wait()
        pltpu.sync_copy(rows[j % 2].at[:, pl.ds(0, _EMBED)],
                        out_hbm.at[pl.ds(base + j * _CHUNK, _CHUNK)])


def _sc_gather(idx3, embed):
    mesh = plsc.VectorSubcoreMesh(core_axis_name="c", subcore_axis_name="s")
    fn = pl.kernel(
        _gather_body,
        mesh=mesh,
        out_type=jax.ShapeDtypeStruct((_TOTAL, _EMBED), jnp.float32),
        scratch_types=[
            pltpu.VMEM((_NCHUNK, _CHUNK), jnp.int32),
            pltpu.VMEM((_CHUNK, 128), jnp.float32),
            pltpu.VMEM((_CHUNK, 128), jnp.float32),
            pltpu.SemaphoreType.DMA,
            pltpu.SemaphoreType.DMA,
        ],
        compiler_params=pltpu.CompilerParams(use_tc_tiling_on_sc=False),
    )
    return fn(idx3, embed)


def _mlp_body(x_ref, w1_ref, b1_ref, w2_ref, b2_ref, w3_ref, b3_ref, y_ref):
    x = x_ref[...]
    h = lax.dot_general(x, w1_ref[...], (((1,), (1,)), ((), ())),
                        preferred_element_type=jnp.float32)
    h = jnp.maximum(h + b1_ref[...], 0.0)
    h = lax.dot_general(h, w2_ref[...], (((1,), (1,)), ((), ())),
                        preferred_element_type=jnp.float32)
    h = jnp.maximum(h + b2_ref[...], 0.0)
    y = lax.dot_general(w3_ref[...], h, (((1,), (1,)), ((), ())),
                        preferred_element_type=jnp.float32)
    y_ref[...] = y + b3_ref[0, 0]


def _tc_mlp(x, W1, b1, W2, b2, W3, b3):
    bb = 1024
    grid = (_BATCH // bb,)
    return pl.pallas_call(
        _mlp_body,
        grid=grid,
        in_specs=[
            pl.BlockSpec((bb, _MAXLEN * _EMBED), lambda i: (i, 0)),
            pl.BlockSpec((_HIDDEN, _MAXLEN * _EMBED), lambda i: (0, 0)),
            pl.BlockSpec((1, _HIDDEN), lambda i: (0, 0)),
            pl.BlockSpec((_HIDDEN, _HIDDEN), lambda i: (0, 0)),
            pl.BlockSpec((1, _HIDDEN), lambda i: (0, 0)),
            pl.BlockSpec((_OUT, _HIDDEN), lambda i: (0, 0)),
            pl.BlockSpec(memory_space=pltpu.SMEM),
        ],
        out_specs=pl.BlockSpec((_OUT, bb), lambda i: (0, i)),
        out_shape=jax.ShapeDtypeStruct((_OUT, _BATCH), jnp.float32),
    )(x, W1, b1, W2, b2, W3, b3)


def kernel(input, embed, W1, b1, W2, b2, W3, b3):
    idx3 = input.reshape(_NW, _NCHUNK, _CHUNK)
    table = _tc_twiden(embed.T)
    rows = _sc_gather(idx3, table)
    x = rows.reshape(_BATCH, _MAXLEN * _EMBED)
    y = _tc_mlp(x, W1, b1.reshape(1, -1), W2, b2.reshape(1, -1),
                W3, b3.reshape(1, -1))
    return y.reshape(_BATCH, _OUT)


# speedup vs baseline: 3.7125x; 1.0174x over previous
import jax
import jax.numpy as jnp
from jax import lax
from jax.experimental import pallas as pl
from jax.experimental.pallas import tpu as pltpu
from jax.experimental.pallas import tpu_sc as plsc

_VOCAB = 1000000
_EMBED = 32
_MAXLEN = 50
_HIDDEN = 128
_OUT = 1
_BATCH = 4096

_NC = 2
_NS = 16
_NW = _NC * _NS
_TOTAL = _BATCH * _MAXLEN
_PER_W = _TOTAL // _NW
_CHUNK = 128
_NCHUNK = _PER_W // _CHUNK

_TBLK = 32768


def _twiden_body(eT_ref, o_ref):
    eye = jnp.eye(_EMBED, dtype=jnp.float32)
    o_ref[:, 0:_EMBED] = lax.dot_general(
        eT_ref[...], eye, (((0,), (0,)), ((), ())),
        preferred_element_type=jnp.float32)


def _tc_twiden(embT):
    grid = (-(-_VOCAB // _TBLK),)
    return pl.pallas_call(
        _twiden_body,
        grid=grid,
        in_specs=[pl.BlockSpec((_EMBED, _TBLK), lambda i: (0, i))],
        out_specs=pl.BlockSpec((_TBLK, 128), lambda i: (i, 0)),
        out_shape=jax.ShapeDtypeStruct((_VOCAB, 128), jnp.float32),
    )(embT)


def _gather_body(idx_hbm, embed_hbm, out_hbm, idx_v, rows0, rows1,
                 sem0, sem1):
    wid = lax.axis_index("s") * _NC + lax.axis_index("c")
    base = wid * _PER_W
    pltpu.sync_copy(idx_hbm.at[wid], idx_v)
    rows = (rows0, rows1)
    sems = (sem0, sem1)

    pltpu.async_copy(embed_hbm.at[idx_v.at[0]], rows[0], sems[0])
    for j in range(_NCHUNK):
        if j + 1 < _NCHUNK:
            pltpu.async_copy(embed_hbm.at[idx_v.at[j + 1]],
                             rows[(j + 1) % 2], sems[(j + 1) % 2])
        pltpu.make_async_copy(embed_hbm.at[idx_v.at[j]], rows[j % 2],
                              sems[j % 2]).wait()
        pltpu.sync_copy(rows[j % 2].at[:, pl.ds(0, _EMBED)],
                        out_hbm.at[pl.ds(base + j * _CHUNK, _CHUNK)])


def _sc_gather(idx3, embed):
    mesh = plsc.VectorSubcoreMesh(core_axis_name="c", subcore_axis_name="s")
    fn = pl.kernel(
        _gather_body,
        mesh=mesh,
        out_type=jax.ShapeDtypeStruct((_TOTAL, _EMBED), jnp.float32),
        scratch_types=[
            pltpu.VMEM((_NCHUNK, _CHUNK), jnp.int32),
            pltpu.VMEM((_CHUNK, 128), jnp.float32),
            pltpu.VMEM((_CHUNK, 128), jnp.float32),
            pltpu.SemaphoreType.DMA,
            pltpu.SemaphoreType.DMA,
        ],
        compiler_params=pltpu.CompilerParams(use_tc_tiling_on_sc=False),
    )
    return fn(idx3, embed)


def _mlp_body(x_ref, w1_ref, b1_ref, w2_ref, b2_ref, w3_ref, b3_ref, y_ref):
    x = x_ref[...]
    h = lax.dot_general(x, w1_ref[...], (((1,), (1,)), ((), ())),
                        preferred_element_type=jnp.float32)
    h = jnp.maximum(h + b1_ref[...], 0.0)
    h = lax.dot_general(h, w2_ref[...], (((1,), (1,)), ((), ())),
                        preferred_element_type=jnp.float32)
    h = jnp.maximum(h + b2_ref[...], 0.0)
    y = lax.dot_general(w3_ref[...], h, (((1,), (1,)), ((), ())),
                        preferred_element_type=jnp.float32)
    y_ref[...] = y + b3_ref[0, 0]


def _tc_mlp(x, W1, b1, W2, b2, W3, b3):
    bb = 1024
    grid = (_BATCH // bb,)
    return pl.pallas_call(
        _mlp_body,
        grid=grid,
        in_specs=[
            pl.BlockSpec((bb, _MAXLEN * _EMBED), lambda i: (i, 0)),
            pl.BlockSpec((_HIDDEN, _MAXLEN * _EMBED), lambda i: (0, 0)),
            pl.BlockSpec((1, _HIDDEN), lambda i: (0, 0)),
            pl.BlockSpec((_HIDDEN, _HIDDEN), lambda i: (0, 0)),
            pl.BlockSpec((1, _HIDDEN), lambda i: (0, 0)),
            pl.BlockSpec((_OUT, _HIDDEN), lambda i: (0, 0)),
            pl.BlockSpec(memory_space=pltpu.SMEM),
        ],
        out_specs=pl.BlockSpec((_OUT, bb), lambda i: (0, i)),
        out_shape=jax.ShapeDtypeStruct((_OUT, _BATCH), jnp.float32),
    )(x, W1, b1, W2, b2, W3, b3)


def kernel(input, embed, W1, b1, W2, b2, W3, b3):
    idx3 = input.reshape(_NW, _NCHUNK, _CHUNK)
    table = _tc_twiden(embed.T)
    rows = _sc_gather(idx3, table)
    x = rows.reshape(_BATCH, _MAXLEN * _EMBED)
    y = _tc_mlp(x, W1, b1.reshape(1, -1), W2, b2.reshape(1, -1),
                W3, b3.reshape(1, -1))
    return y.reshape(_BATCH, _OUT)
